# baseline (speedup 1.0000x reference)
import jax
import jax.numpy as jnp
from jax.experimental import pallas as pl
from jax.experimental.pallas import tpu as pltpu

_BLK = 256


def _vq_kernel(x_ref, wt_ref, cbt_ref, out_ref):
    h = jnp.dot(x_ref[...], wt_ref[...], preferred_element_type=jnp.float32)
    mean = jnp.mean(h, axis=-1, keepdims=True)
    hc = h - mean
    var = jnp.mean(hc * hc, axis=-1, keepdims=True)
    hn = hc * jax.lax.rsqrt(var + 1e-5)
    scores = jnp.dot(hn, cbt_ref[...], preferred_element_type=jnp.float32)
    cbt = cbt_ref[...]
    c2 = jnp.sum(cbt * cbt, axis=0, keepdims=True)
    val = c2 - 2.0 * scores
    m = jnp.min(val, axis=-1, keepdims=True)
    K = val.shape[-1]
    iota = jax.lax.broadcasted_iota(jnp.int32, val.shape, 1)
    idx = jnp.min(jnp.where(val == m, iota, K), axis=-1)
    out_ref[0, 0, :] = idx.astype(jnp.int32)


@jax.jit
def kernel(x, W, codebook):
    B, L, DIM = x.shape
    K, CD = codebook.shape
    N = B * L
    xf = x.reshape(N, DIM)
    wt = W.T
    cbt = codebook.T
    grid = (N // _BLK,)
    out = pl.pallas_call(
        _vq_kernel,
        grid=grid,
        in_specs=[
            pl.BlockSpec((_BLK, DIM), lambda i: (i, 0)),
            pl.BlockSpec((DIM, CD), lambda i: (0, 0)),
            pl.BlockSpec((CD, K), lambda i: (0, 0)),
        ],
        out_specs=pl.BlockSpec((1, 1, _BLK), lambda i: (i, 0, 0)),
        out_shape=jax.ShapeDtypeStruct((N // _BLK, 1, _BLK), jnp.int32),
        compiler_params=pltpu.CompilerParams(
            dimension_semantics=("parallel",)),
    )(xf, wt, cbt)
    return out.reshape(B, L)


# speedup vs baseline: 1.2422x; 1.2422x over previous
import jax
import jax.numpy as jnp
from jax.experimental import pallas as pl
from jax.experimental.pallas import tpu as pltpu

_BLK = 2048


def _vq_kernel(x_ref, wt_ref, cbt_ref, out_ref, c2_ref):
    @pl.when(pl.program_id(0) == 0)
    def _():
        cbt = cbt_ref[...]
        c2_ref[...] = jnp.sum(cbt * cbt, axis=0, keepdims=True)

    h = jnp.dot(x_ref[...], wt_ref[...], preferred_element_type=jnp.float32)
    mean = jnp.mean(h, axis=-1, keepdims=True)
    hc = h - mean
    var = jnp.mean(hc * hc, axis=-1, keepdims=True)
    hn = hc * jax.lax.rsqrt(var + 1e-5)
    scores = jnp.dot(hn, cbt_ref[...], preferred_element_type=jnp.float32)
    val = c2_ref[...] - 2.0 * scores
    m = jnp.min(val, axis=-1, keepdims=True)
    K = val.shape[-1]
    iota = jax.lax.broadcasted_iota(jnp.int32, val.shape, 1)
    idx = jnp.min(jnp.where(val == m, iota, K), axis=-1)
    out_ref[0, 0, :] = idx.astype(jnp.int32)


@jax.jit
def kernel(x, W, codebook):
    B, L, DIM = x.shape
    K, CD = codebook.shape
    N = B * L
    xf = x.reshape(N, DIM)
    wt = W.T
    cbt = codebook.T
    grid = (N // _BLK,)
    out = pl.pallas_call(
        _vq_kernel,
        grid=grid,
        in_specs=[
            pl.BlockSpec((_BLK, DIM), lambda i: (i, 0)),
            pl.BlockSpec((DIM, CD), lambda i: (0, 0)),
            pl.BlockSpec((CD, K), lambda i: (0, 0)),
        ],
        out_specs=pl.BlockSpec((1, 1, _BLK), lambda i: (i, 0, 0)),
        out_shape=jax.ShapeDtypeStruct((N // _BLK, 1, _BLK), jnp.int32),
        scratch_shapes=[pltpu.VMEM((1, K), jnp.float32)],
        compiler_params=pltpu.CompilerParams(
            dimension_semantics=("arbitrary",)),
    )(xf, wt, cbt)
    return out.reshape(B, L)


# speedup vs baseline: 1.3728x; 1.1052x over previous
import jax
import jax.numpy as jnp
from jax.experimental import pallas as pl
from jax.experimental.pallas import tpu as pltpu

_BLK = 2048


def _vq_kernel(x_ref, wt_ref, cbt_ref, out_ref, c2_ref):
    @pl.when(pl.program_id(0) == 0)
    def _():
        cbt = cbt_ref[...]
        c2_ref[...] = jnp.sum(cbt * cbt, axis=0, keepdims=True)

    h = jnp.dot(x_ref[...], wt_ref[...], preferred_element_type=jnp.float32)
    mean = jnp.mean(h, axis=-1, keepdims=True)
    hc = h - mean
    var = jnp.mean(hc * hc, axis=-1, keepdims=True)
    hn = hc * jax.lax.rsqrt(var + 1e-5)
    scores = jnp.dot(hn, cbt_ref[...], preferred_element_type=jnp.float32)
    val = c2_ref[...] - 2.0 * scores
    idx = jnp.argmin(val, axis=-1)
    out_ref[0, 0, :] = idx.astype(jnp.int32)


@jax.jit
def kernel(x, W, codebook):
    B, L, DIM = x.shape
    K, CD = codebook.shape
    N = B * L
    xf = x.reshape(N, DIM)
    wt = W.T
    cbt = codebook.T
    grid = (N // _BLK,)
    out = pl.pallas_call(
        _vq_kernel,
        grid=grid,
        in_specs=[
            pl.BlockSpec((_BLK, DIM), lambda i: (i, 0)),
            pl.BlockSpec((DIM, CD), lambda i: (0, 0)),
            pl.BlockSpec((CD, K), lambda i: (0, 0)),
        ],
        out_specs=pl.BlockSpec((1, 1, _BLK), lambda i: (i, 0, 0)),
        out_shape=jax.ShapeDtypeStruct((N // _BLK, 1, _BLK), jnp.int32),
        scratch_shapes=[pltpu.VMEM((1, K), jnp.float32)],
        compiler_params=pltpu.CompilerParams(
            dimension_semantics=("arbitrary",)),
    )(xf, wt, cbt)
    return out.reshape(B, L)


# speedup vs baseline: 1.4819x; 1.0794x over previous
import jax
import jax.numpy as jnp
from jax.experimental import pallas as pl
from jax.experimental.pallas import tpu as pltpu

_BLK = 1024


def _vq_kernel(x_ref, wt_ref, cbt_ref, out_ref, c2_ref):
    @pl.when(pl.program_id(0) == 0)
    def _():
        cbt = cbt_ref[...]
        c2_ref[...] = jnp.sum(cbt * cbt, axis=0, keepdims=True)

    h = jnp.dot(x_ref[...], wt_ref[...], preferred_element_type=jnp.float32)
    mean = jnp.mean(h, axis=-1, keepdims=True)
    hc = h - mean
    var = jnp.mean(hc * hc, axis=-1, keepdims=True)
    hn = hc * jax.lax.rsqrt(var + 1e-5)
    scores = jnp.dot(hn, cbt_ref[...], preferred_element_type=jnp.float32)
    val = c2_ref[...] - 2.0 * scores
    idx = jnp.argmin(val, axis=-1)
    out_ref[0, 0, :] = idx.astype(jnp.int32)


@jax.jit
def kernel(x, W, codebook):
    B, L, DIM = x.shape
    K, CD = codebook.shape
    N = B * L
    xf = x.reshape(N, DIM)
    wt = W.T
    cbt = codebook.T
    grid = (N // _BLK,)
    out = pl.pallas_call(
        _vq_kernel,
        grid=grid,
        in_specs=[
            pl.BlockSpec((_BLK, DIM), lambda i: (i, 0)),
            pl.BlockSpec((DIM, CD), lambda i: (0, 0)),
            pl.BlockSpec((CD, K), lambda i: (0, 0)),
        ],
        out_specs=pl.BlockSpec((1, 1, _BLK), lambda i: (i, 0, 0)),
        out_shape=jax.ShapeDtypeStruct((N // _BLK, 1, _BLK), jnp.int32),
        scratch_shapes=[pltpu.VMEM((1, K), jnp.float32)],
        compiler_params=pltpu.CompilerParams(
            dimension_semantics=("arbitrary",)),
    )(xf, wt, cbt)
    return out.reshape(B, L)
